# Initial kernel scaffold; baseline (speedup 1.0000x reference)
import jax
import jax.numpy as jnp
from jax.experimental import pallas as pl


def kernel(predictions, anchors):
    raise NotImplementedError("write your pallas kernel here")


# speedup vs baseline: 9.8578x; 9.8578x over previous
import functools

import jax
import jax.numpy as jnp
from jax import lax
from jax.experimental import pallas as pl
from jax.experimental.pallas import tpu as pltpu

N_ANCHORS = 76725
N_PAD = 76800
ROWS = 600
TOP_K = 1000
MAX_DET = 100
IOU_THR = 0.5
SCORE_THR = 0.5
IMG_W = 640.0
IMG_H = 640.0
NEG_INF = float("-inf")
INT_MIN = -2147483648


def _nms_kernel(pred_ref, anc_ref, boxes_out, misc_out,
                x1_s, y1_s, x2_s, y2_s, ar_s, idf_s, w_s):
    f32 = jnp.float32
    i32 = jnp.int32

    ridx = lax.broadcasted_iota(i32, (ROWS, 128), 0)
    cidx = lax.broadcasted_iota(i32, (ROWS, 128), 1)
    gidx = ridx * 128 + cidx

    c0 = pred_ref[4]
    c1 = pred_ref[5]
    score = jnp.maximum(c0, c1) + 0.0
    idf = (c1 > c0).astype(f32)

    ibits = lax.bitcast_convert_type(score, i32)
    key = jnp.where(ibits >= 0, ibits, ibits ^ jnp.int32(0x7FFFFFFF))
    key = jnp.where(gidx < N_ANCHORS, key, jnp.int32(INT_MIN))

    b0 = pred_ref[0] * f32(0.1)
    b1 = pred_ref[1] * f32(0.1)
    b2 = pred_ref[2] * f32(0.2)
    b3 = pred_ref[3] * f32(0.2)
    ax = anc_ref[0]
    ay = anc_ref[1]
    aw = anc_ref[2]
    ah = anc_ref[3]
    x = b0 * aw + ax
    y = b1 * ah + ay
    w = jnp.exp(b2) * aw
    h = jnp.exp(b3) * ah
    x1 = jnp.clip(x - w / 2.0, 0.0, IMG_W)
    y1 = jnp.clip(y - h / 2.0, 0.0, IMG_H)
    x2 = jnp.clip(x + w / 2.0, 0.0, IMG_W)
    y2 = jnp.clip(y + h / 2.0, 0.0, IMG_H)

    def count_ge(t):
        return jnp.sum((key >= t).astype(i32))

    cur0 = jnp.where(count_ge(jnp.int32(0)) >= TOP_K,
                     jnp.int32(0), jnp.int32(INT_MIN))

    def bit_step(t, cur):
        cand = cur + (jnp.int32(1) << (jnp.int32(30) - t))
        return jnp.where(count_ge(cand) >= TOP_K, cand, cur)

    t_key = lax.fori_loop(0, 31, bit_step, cur0)
    m_gt = jnp.sum((key > t_key).astype(i32))
    r_need = TOP_K - m_gt

    eq = key == t_key

    def idx_step(_, lohi):
        lo, hi = lohi
        mid = (lo + hi) // 2
        cnt = jnp.sum((eq & (gidx <= mid)).astype(i32))
        p = cnt >= r_need
        return jnp.where(p, lo, mid + 1), jnp.where(p, mid, hi)

    lo, hi = lax.fori_loop(0, 17, idx_step,
                           (jnp.int32(0), jnp.int32(N_PAD - 1)))
    cstar = lo

    selected = (key > t_key) | (eq & (gidx <= cstar))
    w0 = jnp.where(selected & (score >= SCORE_THR), score, NEG_INF)

    x1_s[...] = x1
    y1_s[...] = y1
    x2_s[...] = x2
    y2_s[...] = y2
    ar_s[...] = (x2 - x1) * (y2 - y1)
    idf_s[...] = idf
    w_s[...] = w0

    def nms_step(t, _):
        wv = w_s[...]
        m = jnp.max(wv)
        valid = m > NEG_INF
        eqm = wv == m
        i = jnp.min(jnp.where(eqm, gidx, jnp.int32(2**30)))
        em = gidx == i
        bx1 = jnp.max(jnp.where(em, x1_s[...], NEG_INF))
        by1 = jnp.max(jnp.where(em, y1_s[...], NEG_INF))
        bx2 = jnp.max(jnp.where(em, x2_s[...], NEG_INF))
        by2 = jnp.max(jnp.where(em, y2_s[...], NEG_INF))
        bid = jnp.max(jnp.where(em, idf_s[...], NEG_INF))
        bar = jnp.max(jnp.where(em, ar_s[...], NEG_INF))
        xx1 = jnp.maximum(bx1, x1_s[...])
        yy1 = jnp.maximum(by1, y1_s[...])
        xx2 = jnp.minimum(bx2, x2_s[...])
        yy2 = jnp.minimum(by2, y2_s[...])
        inter = jnp.maximum(xx2 - xx1, 0.0) * jnp.maximum(yy2 - yy1, 0.0)
        union = bar + ar_s[...] - inter
        iou = inter / jnp.maximum(union, 1e-8)
        w_s[...] = jnp.where((iou > IOU_THR) | em, NEG_INF, wv)

        vf = valid.astype(jnp.float32)
        row = jnp.concatenate(
            [jnp.where(valid, b, 0.0).reshape(1, 1)
             for b in (bx1, by1, bx2, by2)], axis=1)
        boxes_out[pl.ds(t, 1), :] = row
        mrow = jnp.concatenate([
            jnp.where(valid, bid, -1.0).reshape(1, 1),
            jnp.where(valid, m, 0.0).reshape(1, 1),
            vf.reshape(1, 1),
            jnp.zeros((1, 1), jnp.float32)], axis=1)
        misc_out[pl.ds(t, 1), :] = mrow
        return 0

    lax.fori_loop(0, MAX_DET, nms_step, 0)


@jax.jit
def kernel(predictions, anchors):
    f32 = jnp.float32
    pred_t = jnp.pad(predictions[0].T, ((0, 0), (0, N_PAD - N_ANCHORS)))
    pred_t = pred_t.reshape(6, ROWS, 128)
    anc_t = jnp.pad(anchors.T, ((0, 0), (0, N_PAD - N_ANCHORS)))
    anc_t = anc_t.reshape(4, ROWS, 128)

    boxes, misc = pl.pallas_call(
        _nms_kernel,
        out_shape=[
            jax.ShapeDtypeStruct((128, 4), f32),
            jax.ShapeDtypeStruct((128, 4), f32),
        ],
        scratch_shapes=[pltpu.VMEM((ROWS, 128), f32)] * 7,
    )(pred_t, anc_t)

    det_boxes = boxes[:MAX_DET]
    det_ids = misc[:MAX_DET, 0].astype(jnp.int32)
    det_probs = misc[:MAX_DET, 1]
    det_valid = misc[:MAX_DET, 2] > 0.5
    det_boxes = jnp.where(det_valid[:, None], det_boxes, 0.0)
    return (det_boxes, det_ids, det_probs, det_valid)
